# baseline (speedup 1.0000x reference)
import functools
import jax
import jax.numpy as jnp
from jax import lax
from jax.experimental import pallas as pl
from jax.experimental.pallas import tpu as pltpu
from jax.experimental.pallas import tpu_sc as plsc

B = 32
IN_F = 512
OUT_F = 4096
TOP_K = 64
Q = OUT_F // B
L = 16
NW = 32


def _sc_s_kernel(inp_hbm, psn_hbm, s_hbm, ivm, svm, pvm):
    wid = lax.axis_index("s") * 2 + lax.axis_index("c")
    pltpu.sync_copy(inp_hbm, ivm)
    pltpu.sync_copy(psn_hbm.at[wid], pvm)
    ciota = lax.iota(jnp.int32, L) * IN_F
    zeros16 = jnp.zeros((L,), jnp.float32)

    def _zero(k, _):
        svm[pl.ds(k * L, L)] = zeros16
        return _
    lax.fori_loop(0, B * IN_F // L, _zero, None)

    def _scatter(i0, _):
        jv = pvm[pl.ds(i0 * L, L)]
        for u in range(L):
            j = jv[u]
            i = i0 * L + u
            for h in range(2):
                x = ivm[pl.ds(i * B + h * L, L)]
                u = plsc.bitcast(x, jnp.uint32)
                u = (u + jnp.uint32(0x7FFF) + ((u >> jnp.uint32(16)) & jnp.uint32(1)))
                u = u & jnp.uint32(0xFFFF0000)
                x = plsc.bitcast(u, jnp.float32)
                plsc.addupdate_scatter(svm, [ciota + (h * L * IN_F + j)], x)
        return _
    lax.fori_loop(0, IN_F // L, _scatter, None)

    pltpu.sync_copy(svm, s_hbm.at[wid])


def _sc_s_build(inp_flat, psn):
    kfn = pl.kernel(
        _sc_s_kernel,
        out_type=jax.ShapeDtypeStruct((NW, B * IN_F), jnp.float32),
        mesh=plsc.VectorSubcoreMesh(core_axis_name="c", subcore_axis_name="s",
                                    num_cores=2, num_subcores=16),
        scratch_types=[
            pltpu.VMEM((B * IN_F,), jnp.float32),
            pltpu.VMEM((B * IN_F,), jnp.float32),
            pltpu.VMEM((IN_F,), jnp.int32),
        ],
        compiler_params=pltpu.CompilerParams(needs_layout_passes=False),
    )
    return kfn(inp_flat, psn)


def _tc_kernel(w_ref, s_ref, bias_ref, vals_ref, idx_ref, res_scr):
    g = pl.program_id(0)

    @pl.when(g <= B - 1)
    def _mm():
        c = g
        s = s_ref[:, 0, 0, :]
        w32 = w_ref[0].astype(jnp.bfloat16).astype(jnp.float32)
        rt = jax.lax.dot_general(s, w32, (((1,), (1,)), ((), ())),
                                 precision=jax.lax.Precision.HIGHEST,
                                 preferred_element_type=jnp.float32)
        res_scr[c] = rt + bias_ref[...]

    @pl.when(g == B)
    def _topk():
        cur = res_scr[...]
        posidx = (jax.lax.broadcasted_iota(jnp.int32, (B, B, Q), 2) * B
                  + jax.lax.broadcasted_iota(jnp.int32, (B, B, Q), 1))
        kio = jax.lax.broadcasted_iota(jnp.int32, (B, TOP_K), 1)
        vals_acc = jnp.zeros((B, TOP_K), jnp.float32)
        idx_acc = jnp.zeros((B, TOP_K), jnp.int32)
        neg_inf = jnp.float32(-jnp.inf)
        for k in range(TOP_K):
            m = jnp.max(cur, axis=(1, 2), keepdims=True)
            sel = jnp.min(jnp.where(cur == m, posidx, OUT_F), axis=(1, 2),
                          keepdims=True)
            vals_acc = jnp.where(kio == k, m[:, :, 0], vals_acc)
            idx_acc = jnp.where(kio == k, sel[:, :, 0], idx_acc)
            cur = jnp.where(posidx == sel, neg_inf, cur)
        vals_ref[...] = vals_acc
        idx_ref[...] = idx_acc


@jax.jit
def kernel(input, previously_selected_nodes, weight, bias):
    psn = previously_selected_nodes.astype(jnp.int32)
    s = _sc_s_build(input.T.reshape(-1), psn)
    s4 = s.reshape(B, B, 1, IN_F)
    w3 = weight.reshape(B, Q, IN_F)
    bias_bq = bias.reshape(Q, B).T

    vals, idx = pl.pallas_call(
        _tc_kernel,
        grid=(B + 1,),
        in_specs=[
            pl.BlockSpec((1, Q, IN_F), lambda g: (jnp.minimum(g, B - 1), 0, 0)),
            pl.BlockSpec((B, 1, 1, IN_F), lambda g: (0, jnp.minimum(g, B - 1), 0, 0)),
            pl.BlockSpec((B, Q), lambda g: (0, 0)),
        ],
        out_specs=[
            pl.BlockSpec((B, TOP_K), lambda g: (0, 0)),
            pl.BlockSpec((B, TOP_K), lambda g: (0, 0)),
        ],
        out_shape=[
            jax.ShapeDtypeStruct((B, TOP_K), jnp.float32),
            jax.ShapeDtypeStruct((B, TOP_K), jnp.int32),
        ],
        scratch_shapes=[
            pltpu.VMEM((B, B, Q), jnp.float32),
        ],
    )(w3, s4, bias_bq)

    return vals, idx


# speedup vs baseline: 1.8238x; 1.8238x over previous
import jax
import jax.numpy as jnp
from jax.experimental import pallas as pl
from jax.experimental.pallas import tpu as pltpu

B = 32
IN_F = 512
OUT_F = 4096
TOP_K = 64
Q = OUT_F // B


def _fused_kernel(inp_ref, psn_ref, w_ref, bias_ref, vals_ref, idx_ref,
                  s_scr, res_scr):
    g = pl.program_id(0)

    @pl.when(g == 0)
    def _build_s():
        inp_b = inp_ref[...].astype(jnp.bfloat16)
        iota_j = jax.lax.broadcasted_iota(jnp.int32, (IN_F, IN_F), 0)
        for b in range(B):
            row = psn_ref[b, 0, :]
            ohT = (iota_j == row[None, :]).astype(jnp.bfloat16)
            s = jax.lax.dot_general(inp_b, ohT, (((1,), (1,)), ((), ())),
                                    preferred_element_type=jnp.float32)
            s_scr[:, b, :] = s

    @pl.when(jnp.logical_and(g >= 1, g <= B))
    def _mm():
        c = g - 1
        s = s_scr[c]
        w32 = w_ref[0].astype(jnp.bfloat16).astype(jnp.float32)
        rt = jax.lax.dot_general(s, w32, (((1,), (1,)), ((), ())),
                                 precision=jax.lax.Precision.HIGHEST,
                                 preferred_element_type=jnp.float32)
        res_scr[c] = rt + bias_ref[...]

    @pl.when(g == B + 1)
    def _topk():
        cur = res_scr[...]
        posidx = (jax.lax.broadcasted_iota(jnp.int32, (B, B, Q), 2) * B
                  + jax.lax.broadcasted_iota(jnp.int32, (B, B, Q), 1))
        kio = jax.lax.broadcasted_iota(jnp.int32, (B, TOP_K), 1)
        vals_acc = jnp.zeros((B, TOP_K), jnp.float32)
        idx_acc = jnp.zeros((B, TOP_K), jnp.int32)
        neg_inf = jnp.float32(-jnp.inf)
        for k in range(TOP_K):
            m = jnp.max(cur, axis=(1, 2), keepdims=True)
            sel = jnp.min(jnp.where(cur == m, posidx, OUT_F), axis=(1, 2),
                          keepdims=True)
            vals_acc = jnp.where(kio == k, m[:, :, 0], vals_acc)
            idx_acc = jnp.where(kio == k, sel[:, :, 0], idx_acc)
            cur = jnp.where(posidx == sel, neg_inf, cur)
        vals_ref[...] = vals_acc
        idx_ref[...] = idx_acc


@jax.jit
def kernel(input, previously_selected_nodes, weight, bias):
    psn = previously_selected_nodes.astype(jnp.int32).reshape(B, 1, IN_F)
    w3 = weight.reshape(B, Q, IN_F)
    bias_bq = bias.reshape(Q, B).T

    vals, idx = pl.pallas_call(
        _fused_kernel,
        grid=(B + 2,),
        in_specs=[
            pl.BlockSpec((B, IN_F), lambda g: (0, 0)),
            pl.BlockSpec((B, 1, IN_F), lambda g: (0, 0, 0)),
            pl.BlockSpec((1, Q, IN_F), lambda g: (jnp.maximum(g - 1, 0), 0, 0)),
            pl.BlockSpec((B, Q), lambda g: (0, 0)),
        ],
        out_specs=[
            pl.BlockSpec((B, TOP_K), lambda g: (0, 0)),
            pl.BlockSpec((B, TOP_K), lambda g: (0, 0)),
        ],
        out_shape=[
            jax.ShapeDtypeStruct((B, TOP_K), jnp.float32),
            jax.ShapeDtypeStruct((B, TOP_K), jnp.int32),
        ],
        scratch_shapes=[
            pltpu.VMEM((B, B, IN_F), jnp.float32),
            pltpu.VMEM((B, B, Q), jnp.float32),
        ],
    )(input, psn, w3, bias_bq)

    return vals, idx
